# baseline (speedup 1.0000x reference)
import jax
import jax.numpy as jnp
from jax.experimental import pallas as pl
from jax.experimental.pallas import tpu as pltpu

NUM_CLASSES = 10000
CLASSES_PER_TASK = 1000
CURRENT_TASK = 1
BAND_START = CURRENT_TASK * CLASSES_PER_TASK
BAND_END = BAND_START + CLASSES_PER_TASK

ROWS_PER_BLOCK = 128


def _band_affine_kernel(a_ref, b_ref, x_ref, o_ref):
    o_ref[...] = x_ref[...] * a_ref[...] + b_ref[...]


def kernel(x, alpha, beta):
    m, n = x.shape
    col = jnp.arange(n, dtype=jnp.int32)
    in_band = (col >= BAND_START) & (col < BAND_END)
    a_vec = jnp.where(in_band, alpha[0], jnp.float32(1.0))[None, :]
    b_vec = jnp.where(in_band, beta[0], jnp.float32(0.0))[None, :]
    grid = (m // ROWS_PER_BLOCK,)
    return pl.pallas_call(
        _band_affine_kernel,
        grid=grid,
        in_specs=[
            pl.BlockSpec((1, n), lambda i: (0, 0)),
            pl.BlockSpec((1, n), lambda i: (0, 0)),
            pl.BlockSpec((ROWS_PER_BLOCK, n), lambda i: (i, 0)),
        ],
        out_specs=pl.BlockSpec((ROWS_PER_BLOCK, n), lambda i: (i, 0)),
        out_shape=jax.ShapeDtypeStruct((m, n), x.dtype),
        compiler_params=pltpu.CompilerParams(
            dimension_semantics=("parallel",),
        ),
    )(a_vec, b_vec, x)


# speedup vs baseline: 1.0031x; 1.0031x over previous
import jax
import jax.numpy as jnp
from jax.experimental import pallas as pl
from jax.experimental.pallas import tpu as pltpu

NUM_CLASSES = 10000
CLASSES_PER_TASK = 1000
CURRENT_TASK = 1
BAND_START = CURRENT_TASK * CLASSES_PER_TASK
BAND_END = BAND_START + CLASSES_PER_TASK

ROWS_PER_BLOCK = 256


def _band_affine_kernel(a_ref, b_ref, x_ref, o_ref):
    o_ref[...] = x_ref[...] * a_ref[...] + b_ref[...]


def kernel(x, alpha, beta):
    m, n = x.shape
    col = jnp.arange(n, dtype=jnp.int32)
    in_band = (col >= BAND_START) & (col < BAND_END)
    a_vec = jnp.where(in_band, alpha[0], jnp.float32(1.0))[None, :]
    b_vec = jnp.where(in_band, beta[0], jnp.float32(0.0))[None, :]
    grid = (m // ROWS_PER_BLOCK,)
    return pl.pallas_call(
        _band_affine_kernel,
        grid=grid,
        in_specs=[
            pl.BlockSpec((1, n), lambda i: (0, 0)),
            pl.BlockSpec((1, n), lambda i: (0, 0)),
            pl.BlockSpec((ROWS_PER_BLOCK, n), lambda i: (i, 0)),
        ],
        out_specs=pl.BlockSpec((ROWS_PER_BLOCK, n), lambda i: (i, 0)),
        out_shape=jax.ShapeDtypeStruct((m, n), x.dtype),
        compiler_params=pltpu.CompilerParams(
            dimension_semantics=("parallel",),
        ),
    )(a_vec, b_vec, x)


# speedup vs baseline: 1.2328x; 1.2290x over previous
import jax
import jax.numpy as jnp
from jax.experimental import pallas as pl
from jax.experimental.pallas import tpu as pltpu

NUM_CLASSES = 10000
CLASSES_PER_TASK = 1000
CURRENT_TASK = 1
BAND_START = CURRENT_TASK * CLASSES_PER_TASK
BAND_END = BAND_START + CLASSES_PER_TASK

COL_BLOCK = 256
HULL_START = (BAND_START // COL_BLOCK) * COL_BLOCK
HULL_END = -(-BAND_END // COL_BLOCK) * COL_BLOCK
HULL_BLOCKS = (HULL_END - HULL_START) // COL_BLOCK
ROWS_PER_BLOCK = 512


def _band_affine_kernel(alpha_ref, beta_ref, x_ref, o_ref):
    j = pl.program_id(1)
    xv = x_ref[...]
    col = (HULL_START + j * COL_BLOCK
           + jax.lax.broadcasted_iota(jnp.int32, xv.shape, dimension=1))
    in_band = (col >= BAND_START) & (col < BAND_END)
    o_ref[...] = jnp.where(in_band, xv * alpha_ref[0] + beta_ref[0], xv)


def kernel(x, alpha, beta):
    m, n = x.shape
    grid = (m // ROWS_PER_BLOCK, HULL_BLOCKS)
    block = (ROWS_PER_BLOCK, COL_BLOCK)
    first_block = HULL_START // COL_BLOCK
    return pl.pallas_call(
        _band_affine_kernel,
        grid=grid,
        in_specs=[
            pl.BlockSpec(memory_space=pltpu.SMEM),
            pl.BlockSpec(memory_space=pltpu.SMEM),
            pl.BlockSpec(block, lambda i, j: (i, first_block + j)),
        ],
        out_specs=pl.BlockSpec(block, lambda i, j: (i, first_block + j)),
        out_shape=jax.ShapeDtypeStruct((m, n), x.dtype),
        input_output_aliases={2: 0},
        compiler_params=pltpu.CompilerParams(
            dimension_semantics=("arbitrary", "arbitrary"),
        ),
    )(alpha, beta, x)
